# baseline (speedup 1.0000x reference)
import functools

import jax
import jax.numpy as jnp
from jax import lax
from jax.experimental import pallas as pl
from jax.experimental.pallas import tpu as pltpu
from jax.experimental.pallas import tpu_sc as plsc


def _sc_geometry():
    try:
        info = plsc.get_sparse_core_info()
        return info.num_cores, info.num_subcores
    except Exception:
        return 2, 16


GROUP = 512


def _make_gather(B, S, D, NC, NS):
    NW = NC * NS
    b_per_w = B // NW
    ng = b_per_w // GROUP
    nblk = GROUP // 128
    mesh = plsc.VectorSubcoreMesh(
        core_axis_name="c", subcore_axis_name="s", num_cores=NC, num_subcores=NS
    )

    @functools.partial(
        pl.kernel,
        mesh=mesh,
        out_type=jax.ShapeDtypeStruct((B, D), jnp.float32),
        compiler_params=pltpu.CompilerParams(use_tc_tiling_on_sc=False, needs_layout_passes=False),
        scratch_types=[
            pltpu.VMEM((nblk, 128), jnp.int32),
            pltpu.VMEM((GROUP + 32,), jnp.int32),
            pltpu.VMEM((GROUP + 32,), jnp.int32),
            pltpu.VMEM((GROUP, D), jnp.float32),
            pltpu.VMEM((GROUP, D), jnp.float32),
            pltpu.SemaphoreType.DMA,
        ],
    )
    def k(x_hbm, fix_hbm, grd_hbm, out_hbm, idx_v, pos_v, gidx_v, gfix_v, rows_v, sem):
        wid = lax.axis_index("s") * NC + lax.axis_index("c")
        base = wid * b_per_w
        base_xrow = wid * (b_per_w // 128)
        iota = lax.iota(jnp.int32, 16)

        def group_body(g, _):
            xrow = base_xrow + g * nblk
            pltpu.sync_copy(x_hbm.at[pl.ds(xrow, nblk)], idx_v)

            def pre(r, cnt):
                for kk in range(128 // 16):
                    iv = idx_v[r, pl.ds(kk * 16, 16)]
                    m = iv >= S
                    idx_v[r, pl.ds(kk * 16, 16)] = jnp.where(m, 0, iv)
                    csum = plsc.cumsum(m.astype(jnp.int32))
                    tgt = jnp.where(m, cnt + csum - 1, GROUP + iota)
                    plsc.store_scatter(pos_v, [tgt], r * 128 + kk * 16 + iota)
                    plsc.store_scatter(gidx_v, [tgt], iv - S)
                    cnt = cnt + csum[15]
                return cnt

            cnt = lax.fori_loop(0, nblk, pre, jnp.int32(0))
            gidx_v[pl.ds(cnt, 16)] = jnp.zeros((16,), jnp.int32)

            cps = [
                pltpu.async_copy(
                    fix_hbm.at[idx_v.at[j]],
                    rows_v.at[pl.ds(j * 128, 128)],
                    sem,
                )
                for j in range(nblk)
            ]
            for cp in cps:
                cp.wait()

            nfix = (cnt + 15) // 16

            def fix_gather(j, _):
                pltpu.async_copy(
                    grd_hbm.at[gidx_v.at[pl.ds(j * 16, 16)]],
                    gfix_v.at[pl.ds(j * 16, 16)],
                    sem,
                ).wait()
                return 0

            lax.fori_loop(0, nfix, fix_gather, 0)

            def fix_copy(i, _):
                p = pos_v[pl.ds(i, 16)][0]
                for t in range(D // 16):
                    rows_v[p, pl.ds(t * 16, 16)] = gfix_v[i, pl.ds(t * 16, 16)]
                return 0

            lax.fori_loop(0, cnt, fix_copy, 0)

            pltpu.sync_copy(rows_v, out_hbm.at[pl.ds(base + g * GROUP, GROUP)])
            return 0

        lax.fori_loop(0, ng, group_body, 0)

    return k


def kernel(x, fixed_weight, grad_weight):
    S, D = fixed_weight.shape
    B = x.size
    NC, NS = _sc_geometry()
    x2d = x.reshape(-1).astype(jnp.int32).reshape(-1, 128)
    out = _make_gather(B, S, D, NC, NS)(x2d, fixed_weight, grad_weight)
    return out.reshape(x.shape + (D,))
